# baseline (speedup 1.0000x reference)
import functools
import math

import jax
import jax.numpy as jnp
from jax import lax
from jax.experimental import pallas as pl
from jax.experimental.pallas import tpu as pltpu
from jax.experimental.pallas import tpu_sc as plsc

_NC = 2
_NS = 16
_B = 128
_K = 2


def _make_segsum(n_acc, width, j0, j1):
  rows_per_tile = n_acc // _NS
  mesh = plsc.VectorSubcoreMesh(core_axis_name="c", subcore_axis_name="s")

  @functools.partial(
      pl.kernel,
      out_type=jax.ShapeDtypeStruct((_NC, n_acc, width), jnp.float32),
      mesh=mesh,
      compiler_params=pltpu.CompilerParams(use_tc_tiling_on_sc=False),
      scratch_types=[
          pltpu.VMEM((2, _K, _B), jnp.int32),
          pltpu.VMEM((2, _K, _B), jnp.int32),
          pltpu.VMEM((_K, _B, width), jnp.float32),
          pltpu.VMEM((_K, _B, width), jnp.float32),
          pltpu.VMEM_SHARED((n_acc, width), jnp.float32),
          pltpu.SemaphoreType.DMA,
          pltpu.SemaphoreType.DMA,
          pltpu.SemaphoreType.DMA,
      ],
  )
  def segsum(table, srcj, dstj, zer, out,
             src_v, dst_v, rows_a, rows_b, acc, gsem, ssem_a, ssem_b):
    c = lax.axis_index("c")
    s = lax.axis_index("s")
    pltpu.sync_copy(zer, acc.at[pl.ds(s * rows_per_tile, rows_per_tile)])
    plsc.subcore_barrier()
    base_job = jnp.where(c == 0, s * j0, _NS * j0 + s * j1)
    n_chunks = jnp.where(c == 0, j0 // (2 * _K), j1 // (2 * _K))

    def run_half(t, hi, rows_v, ssem):
      jb = base_job + t * (2 * _K) + hi * _K

      @pl.when(t > 0)
      def _drain_prev():
        for j in range(_K):
          pltpu.make_async_copy(table.at[pl.ds(0, _B)], rows_v.at[j], ssem
                                ).wait()

      pltpu.sync_copy(srcj.at[pl.ds(jb, _K)], src_v.at[hi])
      pltpu.sync_copy(dstj.at[pl.ds(jb, _K)], dst_v.at[hi])
      gd = [pltpu.async_copy(table.at[src_v.at[hi, j]], rows_v.at[j], gsem)
            for j in range(_K)]
      for d in gd:
        d.wait()
      for j in range(_K):
        pltpu.async_copy(rows_v.at[j], acc.at[dst_v.at[hi, j]], ssem,
                         add=True)

    def body(t, carry):
      run_half(t, 0, rows_a, ssem_a)
      run_half(t, 1, rows_b, ssem_b)
      return carry

    lax.fori_loop(0, n_chunks, body, 0)

    @pl.when(n_chunks > 0)
    def _drain_tail():
      for j in range(_K):
        pltpu.make_async_copy(table.at[pl.ds(0, _B)], rows_a.at[j], ssem_a
                              ).wait()
        pltpu.make_async_copy(table.at[pl.ds(0, _B)], rows_b.at[j], ssem_b
                              ).wait()

    plsc.subcore_barrier()
    pltpu.sync_copy(acc.at[pl.ds(s * rows_per_tile, rows_per_tile)],
                    out.at[c, pl.ds(s * rows_per_tile, rows_per_tile)])

  return segsum


def _tc1(x_ref, wl_ref, wr_ref, p1e_ref, r1_ref):
  x = x_ref[...]
  p1 = jnp.dot(x, wl_ref[...], preferred_element_type=jnp.float32)
  r1_ref[...] = jnp.dot(x, wr_ref[...], preferred_element_type=jnp.float32)
  n = x.shape[0]
  col = lax.broadcasted_iota(jnp.int32, (n, 16), 1)
  pad = jnp.where(col == 0, 1.0, 0.0).astype(jnp.float32)
  p1e_ref[...] = jnp.concatenate([p1, pad], axis=1)


_IBN = 1.0 / math.sqrt(1.0 + 1e-5)


def _tc2(parts_ref, r1_ref, wl_ref, wr_ref, b1_ref, g1_ref, be1_ref,
         p2_ref, r2_ref, rdeg_ref):
  n, h = r1_ref.shape
  acc = parts_ref[0] + parts_ref[1]
  seg = acc[:n, :h]
  deg = acc[:n, h:h + 1]
  rdeg = 1.0 / jnp.maximum(deg, 1.0)
  pre = seg * rdeg + b1_ref[...] + r1_ref[...]
  h1 = jnp.maximum(pre * _IBN * g1_ref[...] + be1_ref[...], 0.0)
  p2_ref[...] = jnp.dot(h1, wl_ref[...], preferred_element_type=jnp.float32)
  r2_ref[...] = jnp.dot(h1, wr_ref[...], preferred_element_type=jnp.float32)
  rdeg_ref[...] = rdeg


def _tc3(parts_ref, r2_ref, rdeg_ref, b2_ref, g2_ref, be2_ref,
         wm1_ref, bm1_ref, wm2_ref, bm2_ref, out_ref):
  n, h = r2_ref.shape
  acc = parts_ref[0] + parts_ref[1]
  pre = acc[:n] * rdeg_ref[...] + b2_ref[...] + r2_ref[...]
  h2 = jnp.maximum(pre * _IBN * g2_ref[...] + be2_ref[...], 0.0)
  m = jnp.maximum(
      jnp.dot(h2, wm1_ref[...], preferred_element_type=jnp.float32)
      + bm1_ref[...], 0.0)
  o = jnp.dot(m, wm2_ref[...], preferred_element_type=jnp.float32) + bm2_ref[...]
  out_ref[...] = jax.nn.sigmoid(o)


def kernel(x, edge_index, W_l1, b_l1, W_r1, W_l2, b_l2, W_r2,
           g1, be1, g2, be2, W_m1, b_m1, W_m2, b_m2):
  n, d = x.shape
  h = W_l1.shape[0]
  e = edge_index.shape[1]
  w1 = h + 16
  n_acc = ((n + 1 + 127) // 128) * 128
  nw = _NC * _NS
  unit = nw * _B * 2 * _K
  e_pad = ((e + unit - 1) // unit) * unit
  n_jobs = e_pad // _B
  jtot = n_jobs // _NS

  def split(f):
    g = 2 * _K
    j0 = int(round(f * jtot / g)) * g
    return j0, jtot - j0

  src = edge_index[0].astype(jnp.int32)
  dst = edge_index[1].astype(jnp.int32)
  srcj = jnp.concatenate(
      [src, jnp.zeros((e_pad - e,), jnp.int32)]).reshape(n_jobs, _B)
  dstj = jnp.concatenate(
      [dst, jnp.full((e_pad - e,), n, jnp.int32)]).reshape(n_jobs, _B)

  f32 = jnp.float32
  p1e, r1 = pl.pallas_call(
      _tc1,
      out_shape=[jax.ShapeDtypeStruct((n, w1), f32),
                 jax.ShapeDtypeStruct((n, h), f32)],
  )(x, W_l1.T, W_r1.T)

  seg1 = _make_segsum(n_acc, w1, *split(1.0))
  parts1 = seg1(p1e, srcj, dstj, jnp.zeros((n_acc // _NS, w1), f32))

  p2, r2, rdeg = pl.pallas_call(
      _tc2,
      out_shape=[jax.ShapeDtypeStruct((n, h), f32),
                 jax.ShapeDtypeStruct((n, h), f32),
                 jax.ShapeDtypeStruct((n, 1), f32)],
  )(parts1, r1, W_l2.T, W_r2.T,
    b_l1.reshape(1, h), g1.reshape(1, h), be1.reshape(1, h))

  seg2 = _make_segsum(n_acc, h, *split(1.0))
  parts2 = seg2(p2, srcj, dstj, jnp.zeros((n_acc // _NS, h), f32))

  hm = W_m1.shape[0]
  out2d = pl.pallas_call(
      _tc3,
      out_shape=jax.ShapeDtypeStruct((n, 1), f32),
  )(parts2, r2, rdeg,
    b_l2.reshape(1, h), g2.reshape(1, h), be2.reshape(1, h),
    W_m1.T, b_m1.reshape(1, hm), W_m2.T, b_m2.reshape(1, 1))

  return out2d[:, 0]


# speedup vs baseline: 2.5868x; 2.5868x over previous
import functools
import math

import jax
import jax.numpy as jnp
from jax import lax
from jax.experimental import pallas as pl
from jax.experimental.pallas import tpu as pltpu
from jax.experimental.pallas import tpu_sc as plsc

_NC = 2
_NS = 16
_B = 128


def _make_segsum(n_tbl, n_acc, width, k, j0, j1):
  rows_per_tile = n_acc // _NS
  tbl_per_tile = n_tbl // _NS
  mesh = plsc.VectorSubcoreMesh(core_axis_name="c", subcore_axis_name="s")

  @functools.partial(
      pl.kernel,
      out_type=jax.ShapeDtypeStruct((_NC, n_acc, width), jnp.float32),
      mesh=mesh,
      compiler_params=pltpu.CompilerParams(use_tc_tiling_on_sc=False),
      scratch_types=[
          pltpu.VMEM((2, k, _B), jnp.int32),
          pltpu.VMEM((2, k, _B), jnp.int32),
          pltpu.VMEM((k, _B, width), jnp.float32),
          pltpu.VMEM((k, _B, width), jnp.float32),
          pltpu.VMEM_SHARED((n_tbl, width), jnp.float32),
          pltpu.VMEM_SHARED((n_acc, width), jnp.float32),
          pltpu.SemaphoreType.DMA,
          pltpu.SemaphoreType.DMA,
          pltpu.SemaphoreType.DMA,
      ],
  )
  def segsum(table, srcj, dstj, zer, out,
             src_v, dst_v, rows_a, rows_b, tbl, acc, gsem, ssem_a, ssem_b):
    c = lax.axis_index("c")
    s = lax.axis_index("s")
    pltpu.sync_copy(table.at[pl.ds(s * tbl_per_tile, tbl_per_tile)],
                    tbl.at[pl.ds(s * tbl_per_tile, tbl_per_tile)])
    pltpu.sync_copy(zer, acc.at[pl.ds(s * rows_per_tile, rows_per_tile)])
    plsc.subcore_barrier()
    base_job = jnp.where(c == 0, s * j0, _NS * j0 + s * j1)
    n_chunks = jnp.where(c == 0, j0 // (2 * k), j1 // (2 * k))

    def run_half(t, hi, rows_v, ssem):
      jb = base_job + t * (2 * k) + hi * k

      @pl.when(t > 0)
      def _drain_prev():
        for j in range(k):
          pltpu.make_async_copy(table.at[pl.ds(0, _B)], rows_v.at[j], ssem
                                ).wait()

      pltpu.sync_copy(srcj.at[pl.ds(jb, k)], src_v.at[hi])
      pltpu.sync_copy(dstj.at[pl.ds(jb, k)], dst_v.at[hi])
      gd = [pltpu.async_copy(tbl.at[src_v.at[hi, j]], rows_v.at[j], gsem)
            for j in range(k)]
      for d in gd:
        d.wait()
      for j in range(k):
        pltpu.async_copy(rows_v.at[j], acc.at[dst_v.at[hi, j]], ssem,
                         add=True)

    def body(t, carry):
      run_half(t, 0, rows_a, ssem_a)
      run_half(t, 1, rows_b, ssem_b)
      return carry

    lax.fori_loop(0, n_chunks, body, 0)

    @pl.when(n_chunks > 0)
    def _drain_tail():
      for j in range(k):
        pltpu.make_async_copy(table.at[pl.ds(0, _B)], rows_a.at[j], ssem_a
                              ).wait()
        pltpu.make_async_copy(table.at[pl.ds(0, _B)], rows_b.at[j], ssem_b
                              ).wait()

    plsc.subcore_barrier()
    pltpu.sync_copy(acc.at[pl.ds(s * rows_per_tile, rows_per_tile)],
                    out.at[c, pl.ds(s * rows_per_tile, rows_per_tile)])

  return segsum


def _tc1(x_ref, wl_ref, wr_ref, p1e_ref, r1_ref):
  x = x_ref[...]
  p1 = jnp.dot(x, wl_ref[...], preferred_element_type=jnp.float32)
  r1_ref[...] = jnp.dot(x, wr_ref[...], preferred_element_type=jnp.float32)
  n = x.shape[0]
  col = lax.broadcasted_iota(jnp.int32, (n, 16), 1)
  pad = jnp.where(col == 0, 1.0, 0.0).astype(jnp.float32)
  p1e_ref[...] = jnp.concatenate([p1, pad], axis=1)


_IBN = 1.0 / math.sqrt(1.0 + 1e-5)


def _tc2(parts_ref, r1_ref, wl_ref, wr_ref, b1_ref, g1_ref, be1_ref,
         p2_ref, r2_ref, rdeg_ref):
  n, h = r1_ref.shape
  acc = parts_ref[0] + parts_ref[1]
  seg = acc[:n, :h]
  deg = acc[:n, h:h + 1]
  rdeg = 1.0 / jnp.maximum(deg, 1.0)
  pre = seg * rdeg + b1_ref[...] + r1_ref[...]
  h1 = jnp.maximum(pre * _IBN * g1_ref[...] + be1_ref[...], 0.0)
  p2_ref[...] = jnp.dot(h1, wl_ref[...], preferred_element_type=jnp.float32)
  r2_ref[...] = jnp.dot(h1, wr_ref[...], preferred_element_type=jnp.float32)
  rdeg_ref[...] = rdeg


def _tc3(parts_ref, r2_ref, rdeg_ref, b2_ref, g2_ref, be2_ref,
         wm1_ref, bm1_ref, wm2_ref, bm2_ref, out_ref):
  n, h = r2_ref.shape
  acc = parts_ref[0] + parts_ref[1]
  pre = acc[:n] * rdeg_ref[...] + b2_ref[...] + r2_ref[...]
  h2 = jnp.maximum(pre * _IBN * g2_ref[...] + be2_ref[...], 0.0)
  m = jnp.maximum(
      jnp.dot(h2, wm1_ref[...], preferred_element_type=jnp.float32)
      + bm1_ref[...], 0.0)
  o = jnp.dot(m, wm2_ref[...], preferred_element_type=jnp.float32) + bm2_ref[...]
  out_ref[...] = jax.nn.sigmoid(o)


def kernel(x, edge_index, W_l1, b_l1, W_r1, W_l2, b_l2, W_r2,
           g1, be1, g2, be2, W_m1, b_m1, W_m2, b_m2):
  n, d = x.shape
  h = W_l1.shape[0]
  e = edge_index.shape[1]
  w1 = h + 16
  n_acc = ((n + 1 + 127) // 128) * 128
  nw = _NC * _NS
  src = edge_index[0].astype(jnp.int32)
  dst = edge_index[1].astype(jnp.int32)

  def edge_blocks(k, f):
    unit = nw * _B * 2 * k
    e_pad = ((e + unit - 1) // unit) * unit
    n_jobs = e_pad // _B
    jtot = n_jobs // _NS
    g = 2 * k
    j0 = int(round(f * jtot / g)) * g
    srcj = jnp.concatenate(
        [src, jnp.zeros((e_pad - e,), jnp.int32)]).reshape(n_jobs, _B)
    dstj = jnp.concatenate(
        [dst, jnp.full((e_pad - e,), n, jnp.int32)]).reshape(n_jobs, _B)
    return srcj, dstj, j0, jtot - j0

  f32 = jnp.float32
  p1e, r1 = pl.pallas_call(
      _tc1,
      out_shape=[jax.ShapeDtypeStruct((n, w1), f32),
                 jax.ShapeDtypeStruct((n, h), f32)],
  )(x, W_l1.T, W_r1.T)

  srcj1, dstj1, j0_1, j1_1 = edge_blocks(1, 0.5)
  seg1 = _make_segsum(n, n_acc, w1, 1, j0_1, j1_1)
  parts1 = seg1(p1e, srcj1, dstj1, jnp.zeros((n_acc // _NS, w1), f32))

  p2, r2, rdeg = pl.pallas_call(
      _tc2,
      out_shape=[jax.ShapeDtypeStruct((n, h), f32),
                 jax.ShapeDtypeStruct((n, h), f32),
                 jax.ShapeDtypeStruct((n, 1), f32)],
  )(parts1, r1, W_l2.T, W_r2.T,
    b_l1.reshape(1, h), g1.reshape(1, h), be1.reshape(1, h))

  srcj2, dstj2, j0_2, j1_2 = edge_blocks(2, 0.5)
  seg2 = _make_segsum(n, n_acc, h, 2, j0_2, j1_2)
  parts2 = seg2(p2, srcj2, dstj2, jnp.zeros((n_acc // _NS, h), f32))

  hm = W_m1.shape[0]
  out2d = pl.pallas_call(
      _tc3,
      out_shape=jax.ShapeDtypeStruct((n, 1), f32),
  )(parts2, r2, rdeg,
    b_l2.reshape(1, h), g2.reshape(1, h), be2.reshape(1, h),
    W_m1.T, b_m1.reshape(1, hm), W_m2.T, b_m2.reshape(1, 1))

  return out2d[:, 0]


# speedup vs baseline: 2.7605x; 1.0672x over previous
import functools
import math

import jax
import jax.numpy as jnp
from jax import lax
from jax.experimental import pallas as pl
from jax.experimental.pallas import tpu as pltpu
from jax.experimental.pallas import tpu_sc as plsc

_NC = 2
_NS = 16
_B = 128


def _make_segsum(n_tbl, n_acc, width, k, j0, j1, count=False):
  rows_per_tile = n_acc // _NS
  tbl_per_tile = n_tbl // _NS
  mesh = plsc.VectorSubcoreMesh(core_axis_name="c", subcore_axis_name="s")

  out_type = [jax.ShapeDtypeStruct((_NC, n_acc, width), jnp.float32)]
  scratch = [
      pltpu.VMEM((2, k, _B), jnp.int32),
      pltpu.VMEM((2, k, _B), jnp.int32),
      pltpu.VMEM((k, _B, width), jnp.float32),
      pltpu.VMEM((k, _B, width), jnp.float32),
      pltpu.VMEM_SHARED((n_tbl, width), jnp.float32),
      pltpu.VMEM_SHARED((n_acc, width), jnp.float32),
      pltpu.SemaphoreType.DMA,
      pltpu.SemaphoreType.DMA,
      pltpu.SemaphoreType.DMA,
  ]
  if count:
    out_type.append(jax.ShapeDtypeStruct((_NC, n_acc, 16), jnp.float32))
    scratch += [
        pltpu.VMEM((_B, 16), jnp.float32),
        pltpu.VMEM_SHARED((n_acc, 16), jnp.float32),
        pltpu.SemaphoreType.DMA,
    ]

  @functools.partial(
      pl.kernel,
      out_type=out_type,
      mesh=mesh,
      compiler_params=pltpu.CompilerParams(use_tc_tiling_on_sc=False),
      scratch_types=scratch,
  )
  def segsum(table, srcj, dstj, zer, zer16, ones16, out, *rest):
    if count:
      (deg_out, src_v, dst_v, rows_a, rows_b, tbl, acc,
       gsem, ssem_a, ssem_b, ones_v, dacc, dsem) = rest
    else:
      (src_v, dst_v, rows_a, rows_b, tbl, acc,
       gsem, ssem_a, ssem_b) = rest
    c = lax.axis_index("c")
    s = lax.axis_index("s")
    pltpu.sync_copy(table.at[pl.ds(s * tbl_per_tile, tbl_per_tile)],
                    tbl.at[pl.ds(s * tbl_per_tile, tbl_per_tile)])
    pltpu.sync_copy(zer, acc.at[pl.ds(s * rows_per_tile, rows_per_tile)])
    if count:
      pltpu.sync_copy(zer16, dacc.at[pl.ds(s * rows_per_tile, rows_per_tile)])
      pltpu.sync_copy(ones16, ones_v)
    plsc.subcore_barrier()
    base_job = jnp.where(c == 0, s * j0, _NS * j0 + s * j1)
    n_chunks = jnp.where(c == 0, j0 // (2 * k), j1 // (2 * k))

    def drain(rows_v, ssem):
      for j in range(k):
        pltpu.make_async_copy(table.at[pl.ds(0, _B)], rows_v.at[j], ssem
                              ).wait()
        if count:
          pltpu.make_async_copy(zer16.at[pl.ds(0, _B)], ones_v, dsem).wait()

    def run_half(t, hi, rows_v, ssem):
      jb = base_job + t * (2 * k) + hi * k

      @pl.when(t > 0)
      def _drain_prev():
        drain(rows_v, ssem)

      pltpu.sync_copy(srcj.at[pl.ds(jb, k)], src_v.at[hi])
      pltpu.sync_copy(dstj.at[pl.ds(jb, k)], dst_v.at[hi])
      gd = [pltpu.async_copy(tbl.at[src_v.at[hi, j]], rows_v.at[j], gsem)
            for j in range(k)]
      for d in gd:
        d.wait()
      for j in range(k):
        pltpu.async_copy(rows_v.at[j], acc.at[dst_v.at[hi, j]], ssem,
                         add=True)
        if count:
          pltpu.async_copy(ones_v, dacc.at[dst_v.at[hi, j]], dsem, add=True)

    def body(t, carry):
      run_half(t, 0, rows_a, ssem_a)
      run_half(t, 1, rows_b, ssem_b)
      return carry

    lax.fori_loop(0, n_chunks, body, 0)

    @pl.when(n_chunks > 0)
    def _drain_tail():
      drain(rows_a, ssem_a)
      drain(rows_b, ssem_b)

    plsc.subcore_barrier()
    pltpu.sync_copy(acc.at[pl.ds(s * rows_per_tile, rows_per_tile)],
                    out.at[c, pl.ds(s * rows_per_tile, rows_per_tile)])
    if count:
      pltpu.sync_copy(dacc.at[pl.ds(s * rows_per_tile, rows_per_tile)],
                      deg_out.at[c, pl.ds(s * rows_per_tile, rows_per_tile)])

  return segsum


def _tc1(x_ref, wl_ref, wr_ref, p1_ref, r1_ref):
  x = x_ref[...]
  p1_ref[...] = jnp.dot(x, wl_ref[...], preferred_element_type=jnp.float32)
  r1_ref[...] = jnp.dot(x, wr_ref[...], preferred_element_type=jnp.float32)


_IBN = 1.0 / math.sqrt(1.0 + 1e-5)


def _tc2(parts_ref, degp_ref, r1_ref, wl_ref, wr_ref, b1_ref, g1_ref, be1_ref,
         p2_ref, r2_ref, rdeg_ref):
  n, h = r1_ref.shape
  acc = parts_ref[0] + parts_ref[1]
  seg = acc[:n, :h]
  deg = (degp_ref[0] + degp_ref[1])[:n, 0:1]
  rdeg = 1.0 / jnp.maximum(deg, 1.0)
  pre = seg * rdeg + b1_ref[...] + r1_ref[...]
  h1 = jnp.maximum(pre * _IBN * g1_ref[...] + be1_ref[...], 0.0)
  p2_ref[...] = jnp.dot(h1, wl_ref[...], preferred_element_type=jnp.float32)
  r2_ref[...] = jnp.dot(h1, wr_ref[...], preferred_element_type=jnp.float32)
  rdeg_ref[...] = rdeg


def _tc3(parts_ref, r2_ref, rdeg_ref, b2_ref, g2_ref, be2_ref,
         wm1_ref, bm1_ref, wm2_ref, bm2_ref, out_ref):
  n, h = r2_ref.shape
  acc = parts_ref[0] + parts_ref[1]
  pre = acc[:n] * rdeg_ref[...] + b2_ref[...] + r2_ref[...]
  h2 = jnp.maximum(pre * _IBN * g2_ref[...] + be2_ref[...], 0.0)
  m = jnp.maximum(
      jnp.dot(h2, wm1_ref[...], preferred_element_type=jnp.float32)
      + bm1_ref[...], 0.0)
  o = jnp.dot(m, wm2_ref[...], preferred_element_type=jnp.float32) + bm2_ref[...]
  out_ref[...] = jax.nn.sigmoid(o)


def kernel(x, edge_index, W_l1, b_l1, W_r1, W_l2, b_l2, W_r2,
           g1, be1, g2, be2, W_m1, b_m1, W_m2, b_m2):
  n, d = x.shape
  h = W_l1.shape[0]
  e = edge_index.shape[1]
  n_acc = ((n + 1 + 127) // 128) * 128
  nw = _NC * _NS
  src = edge_index[0].astype(jnp.int32)
  dst = edge_index[1].astype(jnp.int32)

  def edge_blocks(k, f):
    unit = nw * _B * 2 * k
    e_pad = ((e + unit - 1) // unit) * unit
    n_jobs = e_pad // _B
    jtot = n_jobs // _NS
    g = 2 * k
    j0 = int(round(f * jtot / g)) * g
    srcj = jnp.concatenate(
        [src, jnp.zeros((e_pad - e,), jnp.int32)]).reshape(n_jobs, _B)
    dstj = jnp.concatenate(
        [dst, jnp.full((e_pad - e,), n, jnp.int32)]).reshape(n_jobs, _B)
    return srcj, dstj, j0, jtot - j0

  f32 = jnp.float32
  p1, r1 = pl.pallas_call(
      _tc1,
      out_shape=[jax.ShapeDtypeStruct((n, h), f32),
                 jax.ShapeDtypeStruct((n, h), f32)],
  )(x, W_l1.T, W_r1.T)

  srcj, dstj, j0, j1 = edge_blocks(2, 0.5)
  zer = jnp.zeros((n_acc // _NS, h), f32)
  zer16 = jnp.zeros((n_acc // _NS, 16), f32)
  ones16 = jnp.ones((_B, 16), f32)

  seg1 = _make_segsum(n, n_acc, h, 2, j0, j1, count=True)
  parts1, degp = seg1(p1, srcj, dstj, zer, zer16, ones16)

  p2, r2, rdeg = pl.pallas_call(
      _tc2,
      out_shape=[jax.ShapeDtypeStruct((n, h), f32),
                 jax.ShapeDtypeStruct((n, h), f32),
                 jax.ShapeDtypeStruct((n, 1), f32)],
  )(parts1, degp, r1, W_l2.T, W_r2.T,
    b_l1.reshape(1, h), g1.reshape(1, h), be1.reshape(1, h))

  seg2 = _make_segsum(n, n_acc, h, 2, j0, j1, count=False)
  (parts2,) = seg2(p2, srcj, dstj, zer, zer16, ones16)

  hm = W_m1.shape[0]
  out2d = pl.pallas_call(
      _tc3,
      out_shape=jax.ShapeDtypeStruct((n, 1), f32),
  )(parts2, r2, rdeg,
    b_l2.reshape(1, h), g2.reshape(1, h), be2.reshape(1, h),
    W_m1.T, b_m1.reshape(1, hm), W_m2.T, b_m2.reshape(1, 1))

  return out2d[:, 0]
